# baseline (speedup 1.0000x reference)
import functools
import math

import jax
import jax.numpy as jnp
import numpy as np
from jax import lax
from jax.experimental import pallas as pl
from jax.experimental.pallas import tpu as pltpu
from jax.experimental.pallas import tpu_sc as plsc

VOCAB = 1000000
EMB = 64
MAX_LEN = 512
B = 4096
L = 200

NC, NS = 2, 16
NW = NC * NS
LANES = 16

VB = VOCAB // 128
VB_MAIN = (VB // NW) * NW
BPW = VB_MAIN // NW
N_EXTRA = VB - VB_MAIN
PACKED = VOCAB // 2


def _make_pe():
    pe = np.zeros((MAX_LEN, EMB), dtype=np.float32)
    position = np.arange(0, MAX_LEN, dtype=np.float32)[:, None]
    div_term = np.exp(
        np.arange(0, EMB, 2, dtype=np.float32) * -(math.log(10000.0) / EMB))
    pe[:, 0::2] = np.sin(position * div_term)
    pe[:, 1::2] = np.cos(position * div_term)
    return np.concatenate([pe[:L], pe[:L]], axis=1)


_PE2 = _make_pe()


def _iota16():
    return lax.iota(jnp.int32, 16)


def _transpose_block(tbuf, obuf, nrows, col_off=0):
    rowvs = [_iota16() + 16 * c for c in range(4)]

    @pl.loop(0, nrows)
    def _row(j):
        for h in range(2):
            cvec = jnp.full((16,), col_off + 2 * j + h, jnp.int32)
            for c in range(4):
                g = plsc.load_gather(tbuf, [rowvs[c], cvec])
                obuf[j, pl.ds(64 * h + 16 * c, 16)] = g


def _pack_body(tableT, aux, t128, tb0, tb1, ob0, ob1, is0, is1, os0, os1):
    wid = lax.axis_index("s") * NC + lax.axis_index("c")
    base = wid * BPW
    tbufs, obufs = [tb0, tb1], [ob0, ob1]
    isems, osems = [is0, is1], [os0, os1]

    def in_src(t):
        return tableT.at[:, pl.ds((base + t) * 128, 128)]

    def out_dst(t):
        return t128.at[pl.ds((base + t) * 64, 64)]

    for s in range(2):
        pltpu.async_copy(in_src(s), tbufs[s], isems[s])

    @pl.loop(0, BPW, step=2)
    def _outer(o):
        for s in range(2):
            t = o + s
            pltpu.make_async_copy(in_src(t), tbufs[s], isems[s]).wait()

            @pl.when(o > 0)
            def _wait_prev():
                pltpu.make_async_copy(obufs[s], out_dst(t - 2), osems[s]).wait()

            _transpose_block(tbufs[s], obufs[s], 64)
            pltpu.async_copy(obufs[s], out_dst(t), osems[s])

            @pl.when(t + 2 < BPW)
            def _next():
                pltpu.async_copy(in_src(t + 2), tbufs[s], isems[s])

    for s in range(2):
        pltpu.make_async_copy(obufs[s], out_dst(BPW - 2 + s), osems[s]).wait()

    @pl.when(wid < N_EXTRA)
    def _extra():
        blk = VB_MAIN + wid
        pltpu.sync_copy(tableT.at[:, pl.ds(blk * 128, 128)], tb0)
        _transpose_block(tb0, ob0, 64)
        pltpu.sync_copy(ob0, t128.at[pl.ds(blk * 64, 64)])

    @pl.when(wid == NW - 1)
    def _tail():
        pltpu.sync_copy(aux, tb1)
        _transpose_block(tb1, ob1, 32, col_off=64)
        pltpu.sync_copy(ob1.at[pl.ds(0, 32)], t128.at[pl.ds(VB * 64, 32)])


def _lookup_body(inpT, t128, pe2, outT,
                 idx_all, pe_v, g0, g1, o0, o1, i20, i21, pc0, pc1,
                 gs0, gs1, os0, os1):
    wid = lax.axis_index("s") * NC + lax.axis_index("c")
    gbufs, obufs = [g0, g1], [o0, o1]
    idx2s, pcols = [i20, i21], [pc0, pc1]
    gsems, osems = [gs0, gs1], [os0, os1]

    pltpu.sync_copy(inpT.at[:, pl.ds(wid * 128, 128)], idx_all)
    pltpu.sync_copy(pe2, pe_v)

    def prep(l, s):
        for lb in range(8):
            sl = pl.ds(16 * lb, 16)
            v = idx_all[l, sl]
            idx2s[s][sl] = lax.shift_right_logical(v, 1)
            pcols[s][sl] = (v & 1) * 64

    def out_dst(l):
        return outT.at[l, :, pl.ds(wid * 128, 128)]

    rowvs = [_iota16() + 16 * lb for lb in range(8)]

    def compute(l, s):
        pc = [pcols[s][pl.ds(16 * lb, 16)] for lb in range(8)]

        @pl.loop(0, EMB)
        def _e(e):
            pe_splat = plsc.load_gather(
                pe_v, [jnp.full((16,), l, jnp.int32),
                       jnp.full((16,), e, jnp.int32)])
            for lb in range(8):
                g = plsc.load_gather(gbufs[s], [rowvs[lb], pc[lb] + e])
                obufs[s][e, pl.ds(16 * lb, 16)] = g * 8.0 + pe_splat

    for s in range(2):
        prep(s, s)
        pltpu.async_copy(t128.at[idx2s[s]], gbufs[s], gsems[s])

    @pl.loop(0, L, step=2)
    def _outer(o):
        for s in range(2):
            l = o + s
            pltpu.make_async_copy(t128.at[idx2s[s]], gbufs[s], gsems[s]).wait()

            @pl.when(o > 0)
            def _wait_prev():
                pltpu.make_async_copy(obufs[s], out_dst(l - 2), osems[s]).wait()

            compute(l, s)
            pltpu.async_copy(obufs[s], out_dst(l), osems[s])

            @pl.when(l + 2 < L)
            def _next():
                prep(l + 2, s)
                pltpu.async_copy(t128.at[idx2s[s]], gbufs[s], gsems[s])

    for s in range(2):
        pltpu.make_async_copy(obufs[s], out_dst(L - 2 + s), osems[s]).wait()


@jax.jit
def _sc_call(inpT, table, pe2):
    mesh = plsc.VectorSubcoreMesh(core_axis_name="c", subcore_axis_name="s")
    params = pltpu.CompilerParams(use_tc_tiling_on_sc=True,
                                  needs_layout_passes=False)

    pack = pl.kernel(
        _pack_body,
        out_type=jax.ShapeDtypeStruct((PACKED, 128), jnp.float32),
        mesh=mesh,
        scratch_types=[pltpu.VMEM((64, 128), jnp.float32) for _ in range(4)]
        + [pltpu.SemaphoreType.DMA for _ in range(4)],
        compiler_params=params,
    )
    tableT = jnp.transpose(table)
    aux = lax.slice(tableT, (0, VOCAB - 128), (EMB, VOCAB))
    t128 = pack(tableT, aux)

    lookup = pl.kernel(
        _lookup_body,
        out_type=jax.ShapeDtypeStruct((L, EMB, B), jnp.float32),
        mesh=mesh,
        scratch_types=[
            pltpu.VMEM((L, 128), jnp.int32),
            pltpu.VMEM((L, 128), jnp.float32),
            pltpu.VMEM((128, 128), jnp.float32),
            pltpu.VMEM((128, 128), jnp.float32),
            pltpu.VMEM((EMB, 128), jnp.float32),
            pltpu.VMEM((EMB, 128), jnp.float32),
            pltpu.VMEM((128,), jnp.int32),
            pltpu.VMEM((128,), jnp.int32),
            pltpu.VMEM((128,), jnp.int32),
            pltpu.VMEM((128,), jnp.int32),
        ]
        + [pltpu.SemaphoreType.DMA for _ in range(4)],
        compiler_params=params,
    )
    outT = lookup(inpT, t128, pe2)
    return jnp.transpose(outT, (2, 0, 1))


def kernel(input, table):
    inpT = jnp.transpose(jnp.asarray(input, jnp.int32))
    pe2 = jnp.asarray(_PE2)
    return _sc_call(inpT, table, pe2)


# speedup vs baseline: 1.9666x; 1.9666x over previous
import functools
import math

import jax
import jax.numpy as jnp
import numpy as np
from jax import lax
from jax.experimental import pallas as pl
from jax.experimental.pallas import tpu as pltpu
from jax.experimental.pallas import tpu_sc as plsc

VOCAB = 1000000
EMB = 64
MAX_LEN = 512
B = 4096
L = 200

NC, NS = 2, 16
NW = NC * NS
LANES = 16

VB = VOCAB // 128
VB_MAIN = (VB // NW) * NW
BPW = VB_MAIN // NW
N_EXTRA = VB - VB_MAIN
PACKED = VOCAB // 2


def _make_pe():
    pe = np.zeros((MAX_LEN, EMB), dtype=np.float32)
    position = np.arange(0, MAX_LEN, dtype=np.float32)[:, None]
    div_term = np.exp(
        np.arange(0, EMB, 2, dtype=np.float32) * -(math.log(10000.0) / EMB))
    pe[:, 0::2] = np.sin(position * div_term)
    pe[:, 1::2] = np.cos(position * div_term)
    return np.concatenate([pe[:L], pe[:L]], axis=1)


_PE2 = _make_pe()


def _iota16():
    return lax.iota(jnp.int32, 16)


def _transpose_block(tbuf, obuf, nrows, col_off=0):
    rowvs = [_iota16() + 16 * c for c in range(4)]

    @plsc.parallel_loop(0, nrows, unroll=4)
    def _row(j):
        for h in range(2):
            cvec = jnp.full((16,), col_off + 2 * j + h, jnp.int32)
            for c in range(4):
                g = plsc.load_gather(tbuf, [rowvs[c], cvec])
                obuf[j, pl.ds(64 * h + 16 * c, 16)] = g


def _pack_body(tableT, aux, t128, tb0, tb1, ob0, ob1, is0, is1, os0, os1):
    wid = lax.axis_index("s") * NC + lax.axis_index("c")
    base = wid * BPW
    tbufs, obufs = [tb0, tb1], [ob0, ob1]
    isems, osems = [is0, is1], [os0, os1]

    def in_src(t):
        return tableT.at[:, pl.ds((base + t) * 128, 128)]

    def out_dst(t):
        return t128.at[pl.ds((base + t) * 64, 64)]

    for s in range(2):
        pltpu.async_copy(in_src(s), tbufs[s], isems[s])

    @pl.loop(0, BPW, step=2)
    def _outer(o):
        for s in range(2):
            t = o + s
            pltpu.make_async_copy(in_src(t), tbufs[s], isems[s]).wait()

            @pl.when(o > 0)
            def _wait_prev():
                pltpu.make_async_copy(obufs[s], out_dst(t - 2), osems[s]).wait()

            _transpose_block(tbufs[s], obufs[s], 64)
            pltpu.async_copy(obufs[s], out_dst(t), osems[s])

            @pl.when(t + 2 < BPW)
            def _next():
                pltpu.async_copy(in_src(t + 2), tbufs[s], isems[s])

    for s in range(2):
        pltpu.make_async_copy(obufs[s], out_dst(BPW - 2 + s), osems[s]).wait()

    @pl.when(wid < N_EXTRA)
    def _extra():
        blk = VB_MAIN + wid
        pltpu.sync_copy(tableT.at[:, pl.ds(blk * 128, 128)], tb0)
        _transpose_block(tb0, ob0, 64)
        pltpu.sync_copy(ob0, t128.at[pl.ds(blk * 64, 64)])

    @pl.when(wid == NW - 1)
    def _tail():
        pltpu.sync_copy(aux, tb1)
        _transpose_block(tb1, ob1, 32, col_off=64)
        pltpu.sync_copy(ob1.at[pl.ds(0, 32)], t128.at[pl.ds(VB * 64, 32)])


def _lookup_body(inpT, t128, pe2, outT,
                 idx_all, pe_v, g0, g1, o0, o1, i20, i21, pc0, pc1,
                 gs0, gs1, os0, os1):
    wid = lax.axis_index("s") * NC + lax.axis_index("c")
    gbufs, obufs = [g0, g1], [o0, o1]
    idx2s, pcols = [i20, i21], [pc0, pc1]
    gsems, osems = [gs0, gs1], [os0, os1]

    pltpu.sync_copy(inpT.at[:, pl.ds(wid * 128, 128)], idx_all)
    pltpu.sync_copy(pe2, pe_v)

    def prep(l, s):
        for lb in range(8):
            sl = pl.ds(16 * lb, 16)
            v = idx_all[l, sl]
            idx2s[s][sl] = lax.shift_right_logical(v, 1)
            pcols[s][sl] = (v & 1) * 64

    def out_dst(l):
        return outT.at[l, :, pl.ds(wid * 128, 128)]

    rowvs = [_iota16() + 16 * lb for lb in range(8)]

    def compute(l, s):
        pc = [pcols[s][pl.ds(16 * lb, 16)] for lb in range(8)]

        @plsc.parallel_loop(0, EMB, unroll=4)
        def _e(e):
            pe_splat = plsc.load_gather(
                pe_v, [jnp.full((16,), l, jnp.int32),
                       jnp.full((16,), e, jnp.int32)])
            for lb in range(8):
                g = plsc.load_gather(gbufs[s], [rowvs[lb], pc[lb] + e])
                obufs[s][e, pl.ds(16 * lb, 16)] = g * 8.0 + pe_splat

    for s in range(2):
        prep(s, s)
        pltpu.async_copy(t128.at[idx2s[s]], gbufs[s], gsems[s])

    @pl.loop(0, L, step=2)
    def _outer(o):
        for s in range(2):
            l = o + s
            pltpu.make_async_copy(t128.at[idx2s[s]], gbufs[s], gsems[s]).wait()

            @pl.when(o > 0)
            def _wait_prev():
                pltpu.make_async_copy(obufs[s], out_dst(l - 2), osems[s]).wait()

            compute(l, s)
            pltpu.async_copy(obufs[s], out_dst(l), osems[s])

            @pl.when(l + 2 < L)
            def _next():
                prep(l + 2, s)
                pltpu.async_copy(t128.at[idx2s[s]], gbufs[s], gsems[s])

    for s in range(2):
        pltpu.make_async_copy(obufs[s], out_dst(L - 2 + s), osems[s]).wait()


@jax.jit
def _sc_call(inpT, table, pe2):
    mesh = plsc.VectorSubcoreMesh(core_axis_name="c", subcore_axis_name="s")
    params = pltpu.CompilerParams(use_tc_tiling_on_sc=True,
                                  needs_layout_passes=False)

    pack = pl.kernel(
        _pack_body,
        out_type=jax.ShapeDtypeStruct((PACKED, 128), jnp.float32),
        mesh=mesh,
        scratch_types=[pltpu.VMEM((64, 128), jnp.float32) for _ in range(4)]
        + [pltpu.SemaphoreType.DMA for _ in range(4)],
        compiler_params=params,
    )
    tableT = jnp.transpose(table)
    aux = lax.slice(tableT, (0, VOCAB - 128), (EMB, VOCAB))
    t128 = pack(tableT, aux)

    lookup = pl.kernel(
        _lookup_body,
        out_type=jax.ShapeDtypeStruct((L, EMB, B), jnp.float32),
        mesh=mesh,
        scratch_types=[
            pltpu.VMEM((L, 128), jnp.int32),
            pltpu.VMEM((L, 128), jnp.float32),
            pltpu.VMEM((128, 128), jnp.float32),
            pltpu.VMEM((128, 128), jnp.float32),
            pltpu.VMEM((EMB, 128), jnp.float32),
            pltpu.VMEM((EMB, 128), jnp.float32),
            pltpu.VMEM((128,), jnp.int32),
            pltpu.VMEM((128,), jnp.int32),
            pltpu.VMEM((128,), jnp.int32),
            pltpu.VMEM((128,), jnp.int32),
        ]
        + [pltpu.SemaphoreType.DMA for _ in range(4)],
        compiler_params=params,
    )
    outT = lookup(inpT, t128, pe2)
    return jnp.transpose(outT, (2, 0, 1))


def kernel(input, table):
    inpT = jnp.transpose(jnp.asarray(input, jnp.int32))
    pe2 = jnp.asarray(_PE2)
    return _sc_call(inpT, table, pe2)


# speedup vs baseline: 5.3353x; 2.7129x over previous
import functools
import math

import jax
import jax.numpy as jnp
import numpy as np
from jax import lax
from jax.experimental import pallas as pl
from jax.experimental.pallas import tpu as pltpu
from jax.experimental.pallas import tpu_sc as plsc

VOCAB = 1000000
EMB = 64
MAX_LEN = 512
B = 4096
L = 200

NC, NS = 2, 16
NW = NC * NS
LANES = 16

VB = VOCAB // 128
VB_MAIN = (VB // NW) * NW
BPW = VB_MAIN // NW
N_EXTRA = VB - VB_MAIN
PACKED = VOCAB // 2


def _make_pe():
    pe = np.zeros((MAX_LEN, EMB), dtype=np.float32)
    position = np.arange(0, MAX_LEN, dtype=np.float32)[:, None]
    div_term = np.exp(
        np.arange(0, EMB, 2, dtype=np.float32) * -(math.log(10000.0) / EMB))
    pe[:, 0::2] = np.sin(position * div_term)
    pe[:, 1::2] = np.cos(position * div_term)
    return np.concatenate([pe[:L], pe[:L]], axis=1)


_PE2 = _make_pe()


def _iota16():
    return lax.iota(jnp.int32, 16)


def _transpose_block(tbuf, obuf, nrows, col_off=0):
    iot = _iota16()
    mask = nrows - 1
    kvec = lax.shift_right_logical(iot, 1)

    @plsc.parallel_loop(0, nrows, unroll=4)
    def _d(d):
        jv = (d + kvec) & mask
        for p in range(2):
            hv = (iot + p) & 1
            cols_t = col_off + 2 * jv + hv
            for c in range(4):
                g = plsc.load_gather(tbuf, [iot + 16 * c, cols_t])
                plsc.store_scatter(obuf, [jv, 64 * hv + 16 * c + iot], g)


def _pack_body(tableT, aux, t128, tb0, tb1, ob0, ob1, is0, is1, os0, os1):
    wid = lax.axis_index("s") * NC + lax.axis_index("c")
    base = wid * BPW
    tbufs, obufs = [tb0, tb1], [ob0, ob1]
    isems, osems = [is0, is1], [os0, os1]

    def in_src(t):
        return tableT.at[:, pl.ds((base + t) * 128, 128)]

    def out_dst(t):
        return t128.at[pl.ds((base + t) * 64, 64)]

    for s in range(2):
        pltpu.async_copy(in_src(s), tbufs[s], isems[s])

    @pl.loop(0, BPW, step=2)
    def _outer(o):
        for s in range(2):
            t = o + s
            pltpu.make_async_copy(in_src(t), tbufs[s], isems[s]).wait()

            @pl.when(o > 0)
            def _wait_prev():
                pltpu.make_async_copy(obufs[s], out_dst(t - 2), osems[s]).wait()

            _transpose_block(tbufs[s], obufs[s], 64)
            pltpu.async_copy(obufs[s], out_dst(t), osems[s])

            @pl.when(t + 2 < BPW)
            def _next():
                pltpu.async_copy(in_src(t + 2), tbufs[s], isems[s])

    for s in range(2):
        pltpu.make_async_copy(obufs[s], out_dst(BPW - 2 + s), osems[s]).wait()

    @pl.when(wid < N_EXTRA)
    def _extra():
        blk = VB_MAIN + wid
        pltpu.sync_copy(tableT.at[:, pl.ds(blk * 128, 128)], tb0)
        _transpose_block(tb0, ob0, 64)
        pltpu.sync_copy(ob0, t128.at[pl.ds(blk * 64, 64)])

    @pl.when(wid == NW - 1)
    def _tail():
        pltpu.sync_copy(aux, tb1)
        _transpose_block(tb1, ob1, 32, col_off=64)
        pltpu.sync_copy(ob1.at[pl.ds(0, 32)], t128.at[pl.ds(VB * 64, 32)])


def _lookup_body(inpT, t128, pe2, outT,
                 idx_all, pe_v, g0, g1, o0, o1, i20, i21, pc0, pc1,
                 gs0, gs1, os0, os1):
    wid = lax.axis_index("s") * NC + lax.axis_index("c")
    gbufs, obufs = [g0, g1], [o0, o1]
    idx2s, pcols = [i20, i21], [pc0, pc1]
    gsems, osems = [gs0, gs1], [os0, os1]

    pltpu.sync_copy(inpT.at[:, pl.ds(wid * 128, 128)], idx_all)
    pltpu.sync_copy(pe2, pe_v)

    def prep(l, s):
        for lb in range(8):
            sl = pl.ds(16 * lb, 16)
            v = idx_all[l, sl]
            idx2s[s][sl] = lax.shift_right_logical(v, 1)
            pcols[s][sl] = (v & 1) * 64

    def out_dst(l):
        return outT.at[l, :, pl.ds(wid * 128, 128)]

    rowvs = [_iota16() + 16 * lb for lb in range(8)]

    iot = _iota16()

    def compute(l, s):
        pc = [pcols[s][pl.ds(16 * lb, 16)] for lb in range(8)]
        lsplat = jnp.full((16,), l, jnp.int32)

        @plsc.parallel_loop(0, EMB, unroll=4)
        def _d(d):
            ev = (d + iot) & (EMB - 1)
            pe_d = plsc.load_gather(pe_v, [lsplat, ev])
            for lb in range(8):
                g = plsc.load_gather(gbufs[s], [rowvs[lb], pc[lb] + ev])
                plsc.store_scatter(obufs[s], [ev, 16 * lb + iot],
                                   g * 8.0 + pe_d)

    for s in range(2):
        prep(s, s)
        pltpu.async_copy(t128.at[idx2s[s]], gbufs[s], gsems[s])

    @pl.loop(0, L, step=2)
    def _outer(o):
        for s in range(2):
            l = o + s
            pltpu.make_async_copy(t128.at[idx2s[s]], gbufs[s], gsems[s]).wait()

            @pl.when(o > 0)
            def _wait_prev():
                pltpu.make_async_copy(obufs[s], out_dst(l - 2), osems[s]).wait()

            compute(l, s)
            pltpu.async_copy(obufs[s], out_dst(l), osems[s])

            @pl.when(l + 2 < L)
            def _next():
                prep(l + 2, s)
                pltpu.async_copy(t128.at[idx2s[s]], gbufs[s], gsems[s])

    for s in range(2):
        pltpu.make_async_copy(obufs[s], out_dst(L - 2 + s), osems[s]).wait()


@jax.jit
def _sc_call(inpT, table, pe2):
    mesh = plsc.VectorSubcoreMesh(core_axis_name="c", subcore_axis_name="s")
    params = pltpu.CompilerParams(use_tc_tiling_on_sc=True,
                                  needs_layout_passes=False)

    pack = pl.kernel(
        _pack_body,
        out_type=jax.ShapeDtypeStruct((PACKED, 128), jnp.float32),
        mesh=mesh,
        scratch_types=[pltpu.VMEM((64, 128), jnp.float32) for _ in range(4)]
        + [pltpu.SemaphoreType.DMA for _ in range(4)],
        compiler_params=params,
    )
    tableT = jnp.transpose(table)
    aux = lax.slice(tableT, (0, VOCAB - 128), (EMB, VOCAB))
    t128 = pack(tableT, aux)

    lookup = pl.kernel(
        _lookup_body,
        out_type=jax.ShapeDtypeStruct((L, EMB, B), jnp.float32),
        mesh=mesh,
        scratch_types=[
            pltpu.VMEM((L, 128), jnp.int32),
            pltpu.VMEM((L, 128), jnp.float32),
            pltpu.VMEM((128, 128), jnp.float32),
            pltpu.VMEM((128, 128), jnp.float32),
            pltpu.VMEM((EMB, 128), jnp.float32),
            pltpu.VMEM((EMB, 128), jnp.float32),
            pltpu.VMEM((128,), jnp.int32),
            pltpu.VMEM((128,), jnp.int32),
            pltpu.VMEM((128,), jnp.int32),
            pltpu.VMEM((128,), jnp.int32),
        ]
        + [pltpu.SemaphoreType.DMA for _ in range(4)],
        compiler_params=params,
    )
    outT = lookup(inpT, t128, pe2)
    return jnp.transpose(outT, (2, 0, 1))


def kernel(input, table):
    inpT = jnp.transpose(jnp.asarray(input, jnp.int32))
    pe2 = jnp.asarray(_PE2)
    return _sc_call(inpT, table, pe2)


# speedup vs baseline: 6.3898x; 1.1977x over previous
import functools
import math

import jax
import jax.numpy as jnp
import numpy as np
from jax import lax
from jax.experimental import pallas as pl
from jax.experimental.pallas import tpu as pltpu
from jax.experimental.pallas import tpu_sc as plsc

VOCAB = 1000000
EMB = 64
MAX_LEN = 512
B = 4096
L = 200

NC, NS = 2, 16
NW = NC * NS
LANES = 16

VB = VOCAB // 128
VB_MAIN = (VB // NW) * NW
BPW = VB_MAIN // NW
N_EXTRA = VB - VB_MAIN
PACKED = VOCAB // 2


def _make_pe():
    pe = np.zeros((MAX_LEN, EMB), dtype=np.float32)
    position = np.arange(0, MAX_LEN, dtype=np.float32)[:, None]
    div_term = np.exp(
        np.arange(0, EMB, 2, dtype=np.float32) * -(math.log(10000.0) / EMB))
    pe[:, 0::2] = np.sin(position * div_term)
    pe[:, 1::2] = np.cos(position * div_term)
    return np.concatenate([pe[:L], pe[:L]], axis=1)


_PE2 = _make_pe()


def _iota16():
    return lax.iota(jnp.int32, 16)


def _transpose_block(tbuf, obuf, nrows, col_off=0):
    iot = _iota16()
    mask = nrows - 1
    kvec = lax.shift_right_logical(iot, 1)

    @plsc.parallel_loop(0, nrows, unroll=4)
    def _d(d):
        jv = (d + kvec) & mask
        for p in range(2):
            hv = (iot + p) & 1
            cols_t = col_off + 2 * jv + hv
            for c in range(4):
                g = plsc.load_gather(tbuf, [iot + 16 * c, cols_t])
                plsc.store_scatter(obuf, [jv, 64 * hv + 16 * c + iot], g)


def _pack_body(tableT, aux, t128, tb0, tb1, ob0, ob1, is0, is1, os0, os1):
    wid = lax.axis_index("s") * NC + lax.axis_index("c")
    base = wid * BPW
    tbufs, obufs = [tb0, tb1], [ob0, ob1]
    isems, osems = [is0, is1], [os0, os1]

    def in_src(t):
        return tableT.at[:, pl.ds((base + t) * 128, 128)]

    def out_dst(t):
        return t128.at[pl.ds((base + t) * 64, 64)]

    for s in range(2):
        pltpu.async_copy(in_src(s), tbufs[s], isems[s])

    @pl.loop(0, BPW, step=2)
    def _outer(o):
        for s in range(2):
            t = o + s
            pltpu.make_async_copy(in_src(t), tbufs[s], isems[s]).wait()

            @pl.when(o > 0)
            def _wait_prev():
                pltpu.make_async_copy(obufs[s], out_dst(t - 2), osems[s]).wait()

            _transpose_block(tbufs[s], obufs[s], 64)
            pltpu.async_copy(obufs[s], out_dst(t), osems[s])

            @pl.when(t + 2 < BPW)
            def _next():
                pltpu.async_copy(in_src(t + 2), tbufs[s], isems[s])

    for s in range(2):
        pltpu.make_async_copy(obufs[s], out_dst(BPW - 2 + s), osems[s]).wait()

    @pl.when(wid < N_EXTRA)
    def _extra():
        blk = VB_MAIN + wid
        pltpu.sync_copy(tableT.at[:, pl.ds(blk * 128, 128)], tb0)
        _transpose_block(tb0, ob0, 64)
        pltpu.sync_copy(ob0, t128.at[pl.ds(blk * 64, 64)])

    @pl.when(wid == NW - 1)
    def _tail():
        pltpu.sync_copy(aux, tb1)
        _transpose_block(tb1, ob1, 32, col_off=64)
        pltpu.sync_copy(ob1.at[pl.ds(0, 32)], t128.at[pl.ds(VB * 64, 32)])


def _lookup_body(inpT, t64, pe2, outT,
                 idx_all, pe_v, g0, g1, o0, o1,
                 gs0, gs1, os0, os1):
    wid = lax.axis_index("s") * NC + lax.axis_index("c")
    gbufs, obufs = [g0, g1], [o0, o1]
    gsems, osems = [gs0, gs1], [os0, os1]

    pltpu.sync_copy(inpT.at[:, pl.ds(wid * 128, 128)], idx_all)
    pltpu.sync_copy(pe2, pe_v)

    def out_dst(l):
        return outT.at[l, :, wid]

    def gather(l, s):
        return pltpu.make_async_copy(t64.at[idx_all.at[l]], gbufs[s], gsems[s])

    rowvs = [_iota16() + 16 * lb for lb in range(8)]
    iot = _iota16()

    def compute(l, s):
        lsplat = jnp.full((16,), l, jnp.int32)

        @plsc.parallel_loop(0, EMB, unroll=4)
        def _d(d):
            ev = (d + iot) & (EMB - 1)
            pe_d = plsc.load_gather(pe_v, [lsplat, ev])
            et = lax.shift_right_logical(ev, 3)
            es = ev & 7
            for lb in range(8):
                g = plsc.load_gather(gbufs[s], [rowvs[lb], ev])
                plsc.store_scatter(obufs[s], [et, es, 16 * lb + iot],
                                   g * 8.0 + pe_d)

    for s in range(2):
        gather(s, s).start()

    @pl.loop(0, L, step=2)
    def _outer(o):
        for s in range(2):
            l = o + s
            gather(l, s).wait()

            @pl.when(o > 0)
            def _wait_prev():
                pltpu.make_async_copy(obufs[s], out_dst(l - 2), osems[s]).wait()

            compute(l, s)
            pltpu.async_copy(obufs[s], out_dst(l), osems[s])

            @pl.when(l + 2 < L)
            def _next():
                gather(l + 2, s).start()

    for s in range(2):
        pltpu.make_async_copy(obufs[s], out_dst(L - 2 + s), osems[s]).wait()


@jax.jit
def _sc_call(inpT, table, pe2):
    mesh = plsc.VectorSubcoreMesh(core_axis_name="c", subcore_axis_name="s")
    params = pltpu.CompilerParams(use_tc_tiling_on_sc=True,
                                  needs_layout_passes=False)

    pack = pl.kernel(
        _pack_body,
        out_type=jax.ShapeDtypeStruct((PACKED, 128), jnp.float32),
        mesh=mesh,
        scratch_types=[pltpu.VMEM((64, 128), jnp.float32) for _ in range(4)]
        + [pltpu.SemaphoreType.DMA for _ in range(4)],
        compiler_params=params,
    )
    tableT = jnp.transpose(table)
    aux = lax.slice(tableT, (0, VOCAB - 128), (EMB, VOCAB))
    t128 = pack(tableT, aux)

    lookup = pl.kernel(
        _lookup_body,
        out_type=jax.ShapeDtypeStruct((L, EMB // 8, B // 128, 8, 128),
                                      jnp.float32),
        mesh=mesh,
        scratch_types=[
            pltpu.VMEM((L, 128), jnp.int32),
            pltpu.VMEM((L, 128), jnp.float32),
            pltpu.VMEM((128, EMB), jnp.float32),
            pltpu.VMEM((128, EMB), jnp.float32),
            pltpu.VMEM((8, 8, 128), jnp.float32),
            pltpu.VMEM((8, 8, 128), jnp.float32),
        ]
        + [pltpu.SemaphoreType.DMA for _ in range(4)],
        compiler_params=pltpu.CompilerParams(use_tc_tiling_on_sc=False,
                                             needs_layout_passes=False),
    )
    t64 = jnp.reshape(t128, (VOCAB, EMB))
    out5 = lookup(inpT, t64, pe2)
    return jnp.transpose(out5, (2, 4, 0, 1, 3)).reshape(B, L, EMB)


def kernel(input, table):
    inpT = jnp.transpose(jnp.asarray(input, jnp.int32))
    pe2 = jnp.asarray(_PE2)
    return _sc_call(inpT, table, pe2)


# speedup vs baseline: 7.8541x; 1.2292x over previous
import functools
import math

import jax
import jax.numpy as jnp
import numpy as np
from jax import lax
from jax.experimental import pallas as pl
from jax.experimental.pallas import tpu as pltpu
from jax.experimental.pallas import tpu_sc as plsc

VOCAB = 1000000
EMB = 64
MAX_LEN = 512
B = 4096
L = 200

NC, NS = 2, 16
NW = NC * NS
LANES = 16

VB = VOCAB // 128
VB_MAIN = (VB // NW) * NW
BPW = VB_MAIN // NW
N_EXTRA = VB - VB_MAIN
PACKED = VOCAB // 2


def _make_pe():
    pe = np.zeros((MAX_LEN, EMB), dtype=np.float32)
    position = np.arange(0, MAX_LEN, dtype=np.float32)[:, None]
    div_term = np.exp(
        np.arange(0, EMB, 2, dtype=np.float32) * -(math.log(10000.0) / EMB))
    pe[:, 0::2] = np.sin(position * div_term)
    pe[:, 1::2] = np.cos(position * div_term)
    return np.concatenate([pe[:L], pe[:L]], axis=1)


_PE2 = _make_pe()


def _iota16():
    return lax.iota(jnp.int32, 16)


def _transpose_block(tbuf, obuf, nrows, col_off=0):
    iot = _iota16()
    mask = nrows - 1
    kvec = lax.shift_right_logical(iot, 1)

    @plsc.parallel_loop(0, nrows, unroll=4)
    def _d(d):
        jv = (d + kvec) & mask
        for p in range(2):
            hv = (iot + p) & 1
            cols_t = col_off + 2 * jv + hv
            for c in range(4):
                g = plsc.load_gather(tbuf, [iot + 16 * c, cols_t])
                plsc.store_scatter(obuf, [jv, 64 * hv + 16 * c + iot], g)


PNB = 4


def _pack_body(tableT, aux, t128, *scratch):
    wid = lax.axis_index("s") * NC + lax.axis_index("c")
    base = wid * BPW
    tbufs = list(scratch[0:PNB])
    obufs = list(scratch[PNB:2 * PNB])
    isems = list(scratch[2 * PNB:3 * PNB])
    osems = list(scratch[3 * PNB:4 * PNB])
    tb0, ob0 = tbufs[0], obufs[0]
    tb1, ob1 = tbufs[1], obufs[1]

    def in_src(t):
        return tableT.at[:, pl.ds((base + t) * 128, 128)]

    def out_dst(t):
        return t128.at[pl.ds((base + t) * 64, 64)]

    for s in range(PNB):
        pltpu.async_copy(in_src(s), tbufs[s], isems[s])

    @pl.loop(0, BPW, step=PNB)
    def _outer(o):
        for s in range(PNB):
            t = o + s
            pltpu.make_async_copy(in_src(t), tbufs[s], isems[s]).wait()

            @pl.when(o > 0)
            def _wait_prev():
                pltpu.make_async_copy(
                    obufs[s], out_dst(t - PNB), osems[s]).wait()

            _transpose_block(tbufs[s], obufs[s], 64)
            pltpu.async_copy(obufs[s], out_dst(t), osems[s])

            @pl.when(t + PNB < BPW)
            def _next():
                pltpu.async_copy(in_src(t + PNB), tbufs[s], isems[s])

    for s in range(PNB):
        pltpu.make_async_copy(obufs[s], out_dst(BPW - PNB + s), osems[s]).wait()

    @pl.when(wid < N_EXTRA)
    def _extra():
        blk = VB_MAIN + wid
        pltpu.sync_copy(tableT.at[:, pl.ds(blk * 128, 128)], tb0)
        _transpose_block(tb0, ob0, 64)
        pltpu.sync_copy(ob0, t128.at[pl.ds(blk * 64, 64)])

    @pl.when(wid == NW - 1)
    def _tail():
        pltpu.sync_copy(aux, tb1)
        _transpose_block(tb1, ob1, 32, col_off=64)
        pltpu.sync_copy(ob1.at[pl.ds(0, 32)], t128.at[pl.ds(VB * 64, 32)])


LNB = 4


def _lookup_body(inpT, t64, pe2, outT, idx_all, pe_v, *scratch):
    wid = lax.axis_index("s") * NC + lax.axis_index("c")
    gbufs = list(scratch[0:LNB])
    obufs = list(scratch[LNB:2 * LNB])
    gsems = list(scratch[2 * LNB:3 * LNB])
    osems = list(scratch[3 * LNB:4 * LNB])

    pltpu.sync_copy(inpT.at[:, pl.ds(wid * 128, 128)], idx_all)
    pltpu.sync_copy(pe2, pe_v)

    def out_dst(l):
        return outT.at[l, :, wid]

    def gather(l, s):
        return pltpu.make_async_copy(t64.at[idx_all.at[l]], gbufs[s], gsems[s])

    rowvs = [_iota16() + 16 * lb for lb in range(8)]
    iot = _iota16()

    def compute(l, s):
        lsplat = jnp.full((16,), l, jnp.int32)

        @plsc.parallel_loop(0, EMB, unroll=4)
        def _d(d):
            ev = (d + iot) & (EMB - 1)
            pe_d = plsc.load_gather(pe_v, [lsplat, ev])
            et = lax.shift_right_logical(ev, 3)
            es = ev & 7
            for lb in range(8):
                g = plsc.load_gather(gbufs[s], [rowvs[lb], ev])
                plsc.store_scatter(obufs[s], [et, es, 16 * lb + iot],
                                   g * 8.0 + pe_d)

    for s in range(LNB):
        gather(s, s).start()

    @pl.loop(0, L, step=LNB)
    def _outer(o):
        for s in range(LNB):
            l = o + s
            gather(l, s).wait()

            @pl.when(o > 0)
            def _wait_prev():
                pltpu.make_async_copy(
                    obufs[s], out_dst(l - LNB), osems[s]).wait()

            compute(l, s)
            pltpu.async_copy(obufs[s], out_dst(l), osems[s])

            @pl.when(l + LNB < L)
            def _next():
                gather(l + LNB, s).start()

    for s in range(LNB):
        pltpu.make_async_copy(obufs[s], out_dst(L - LNB + s), osems[s]).wait()


@jax.jit
def _sc_call(inpT, table, pe2):
    mesh = plsc.VectorSubcoreMesh(core_axis_name="c", subcore_axis_name="s")
    params = pltpu.CompilerParams(use_tc_tiling_on_sc=True,
                                  needs_layout_passes=False)

    pack = pl.kernel(
        _pack_body,
        out_type=jax.ShapeDtypeStruct((PACKED, 128), jnp.float32),
        mesh=mesh,
        scratch_types=[pltpu.VMEM((64, 128), jnp.float32)
                       for _ in range(2 * PNB)]
        + [pltpu.SemaphoreType.DMA for _ in range(2 * PNB)],
        compiler_params=params,
    )
    tableT = jnp.transpose(table)
    aux = lax.slice(tableT, (0, VOCAB - 128), (EMB, VOCAB))
    t128 = pack(tableT, aux)

    lookup = pl.kernel(
        _lookup_body,
        out_type=jax.ShapeDtypeStruct((L, EMB // 8, B // 128, 8, 128),
                                      jnp.float32),
        mesh=mesh,
        scratch_types=[
            pltpu.VMEM((L, 128), jnp.int32),
            pltpu.VMEM((L, 128), jnp.float32),
        ]
        + [pltpu.VMEM((128, EMB), jnp.float32) for _ in range(LNB)]
        + [pltpu.VMEM((8, 8, 128), jnp.float32) for _ in range(LNB)]
        + [pltpu.SemaphoreType.DMA for _ in range(2 * LNB)],
        compiler_params=pltpu.CompilerParams(use_tc_tiling_on_sc=False,
                                             needs_layout_passes=False),
    )
    t64 = jnp.reshape(t128, (VOCAB, EMB))
    out5 = lookup(inpT, t64, pe2)
    return jnp.transpose(out5, (2, 4, 0, 1, 3)).reshape(B, L, EMB)


def kernel(input, table):
    inpT = jnp.transpose(jnp.asarray(input, jnp.int32))
    pe2 = jnp.asarray(_PE2)
    return _sc_call(inpT, table, pe2)


# speedup vs baseline: 8.0342x; 1.0229x over previous
import functools
import math

import jax
import jax.numpy as jnp
import numpy as np
from jax import lax
from jax.experimental import pallas as pl
from jax.experimental.pallas import tpu as pltpu
from jax.experimental.pallas import tpu_sc as plsc

VOCAB = 1000000
EMB = 64
MAX_LEN = 512
B = 4096
L = 200

NC, NS = 2, 16
NW = NC * NS
LANES = 16

VB = VOCAB // 128
VB_MAIN = (VB // NW) * NW
BPW = VB_MAIN // NW
N_EXTRA = VB - VB_MAIN
PACKED = VOCAB // 2


def _make_pe():
    pe = np.zeros((MAX_LEN, EMB), dtype=np.float32)
    position = np.arange(0, MAX_LEN, dtype=np.float32)[:, None]
    div_term = np.exp(
        np.arange(0, EMB, 2, dtype=np.float32) * -(math.log(10000.0) / EMB))
    pe[:, 0::2] = np.sin(position * div_term)
    pe[:, 1::2] = np.cos(position * div_term)
    return np.concatenate([pe[:L], pe[:L]], axis=1)


_PE2 = _make_pe()


def _iota16():
    return lax.iota(jnp.int32, 16)


def _transpose_block(tbuf, obuf, nrows, col_off=0):
    iot = _iota16()
    mask = nrows - 1
    kvec = lax.shift_right_logical(iot, 1)

    @plsc.parallel_loop(0, nrows, unroll=8)
    def _d(d):
        jv = (d + kvec) & mask
        for p in range(2):
            hv = (iot + p) & 1
            cols_t = col_off + 2 * jv + hv
            for c in range(4):
                g = plsc.load_gather(tbuf, [iot + 16 * c, cols_t])
                plsc.store_scatter(obuf, [jv, 64 * hv + 16 * c + iot], g)


PNB = 4


def _pack_body(tableT, aux, t128, *scratch):
    wid = lax.axis_index("s") * NC + lax.axis_index("c")
    base = wid * BPW
    tbufs = list(scratch[0:PNB])
    obufs = list(scratch[PNB:2 * PNB])
    isems = list(scratch[2 * PNB:3 * PNB])
    osems = list(scratch[3 * PNB:4 * PNB])
    tb0, ob0 = tbufs[0], obufs[0]
    tb1, ob1 = tbufs[1], obufs[1]

    def in_src(t):
        return tableT.at[:, pl.ds((base + t) * 128, 128)]

    def out_dst(t):
        return t128.at[pl.ds((base + t) * 64, 64)]

    for s in range(PNB):
        pltpu.async_copy(in_src(s), tbufs[s], isems[s])

    @pl.loop(0, BPW, step=PNB)
    def _outer(o):
        for s in range(PNB):
            t = o + s
            pltpu.make_async_copy(in_src(t), tbufs[s], isems[s]).wait()

            @pl.when(o > 0)
            def _wait_prev():
                pltpu.make_async_copy(
                    obufs[s], out_dst(t - PNB), osems[s]).wait()

            _transpose_block(tbufs[s], obufs[s], 64)
            pltpu.async_copy(obufs[s], out_dst(t), osems[s])

            @pl.when(t + PNB < BPW)
            def _next():
                pltpu.async_copy(in_src(t + PNB), tbufs[s], isems[s])

    for s in range(PNB):
        pltpu.make_async_copy(obufs[s], out_dst(BPW - PNB + s), osems[s]).wait()

    @pl.when(wid < N_EXTRA)
    def _extra():
        blk = VB_MAIN + wid
        pltpu.sync_copy(tableT.at[:, pl.ds(blk * 128, 128)], tb0)
        _transpose_block(tb0, ob0, 64)
        pltpu.sync_copy(ob0, t128.at[pl.ds(blk * 64, 64)])

    @pl.when(wid == NW - 1)
    def _tail():
        pltpu.sync_copy(aux, tb1)
        _transpose_block(tb1, ob1, 32, col_off=64)
        pltpu.sync_copy(ob1.at[pl.ds(0, 32)], t128.at[pl.ds(VB * 64, 32)])


LNB = 4


def _lookup_body(inpT, t64, pe2, outT, idx_all, pe_v, *scratch):
    wid = lax.axis_index("s") * NC + lax.axis_index("c")
    gbufs = list(scratch[0:LNB])
    obufs = list(scratch[LNB:2 * LNB])
    gsems = list(scratch[2 * LNB:3 * LNB])
    osems = list(scratch[3 * LNB:4 * LNB])

    pltpu.sync_copy(inpT.at[:, pl.ds(wid * 128, 128)], idx_all)
    pltpu.sync_copy(pe2, pe_v)

    def out_dst(l):
        return outT.at[l, :, wid]

    def gather(l, s):
        return pltpu.make_async_copy(t64.at[idx_all.at[l]], gbufs[s], gsems[s])

    rowvs = [_iota16() + 16 * lb for lb in range(8)]
    iot = _iota16()

    def compute(l, s):
        lsplat = jnp.full((16,), l, jnp.int32)

        @plsc.parallel_loop(0, EMB, unroll=8)
        def _d(d):
            ev = (d + iot) & (EMB - 1)
            pe_d = plsc.load_gather(pe_v, [lsplat, ev])
            et = lax.shift_right_logical(ev, 3)
            es = ev & 7
            for lb in range(8):
                g = plsc.load_gather(gbufs[s], [rowvs[lb], ev])
                plsc.store_scatter(obufs[s], [et, es, 16 * lb + iot],
                                   g * 8.0 + pe_d)

    for s in range(LNB):
        gather(s, s).start()

    @pl.loop(0, L, step=LNB)
    def _outer(o):
        for s in range(LNB):
            l = o + s
            gather(l, s).wait()

            @pl.when(o > 0)
            def _wait_prev():
                pltpu.make_async_copy(
                    obufs[s], out_dst(l - LNB), osems[s]).wait()

            compute(l, s)
            pltpu.async_copy(obufs[s], out_dst(l), osems[s])

            @pl.when(l + LNB < L)
            def _next():
                gather(l + LNB, s).start()

    for s in range(LNB):
        pltpu.make_async_copy(obufs[s], out_dst(L - LNB + s), osems[s]).wait()


@jax.jit
def _sc_call(inpT, table, pe2):
    mesh = plsc.VectorSubcoreMesh(core_axis_name="c", subcore_axis_name="s")
    params = pltpu.CompilerParams(use_tc_tiling_on_sc=True,
                                  needs_layout_passes=False)

    pack = pl.kernel(
        _pack_body,
        out_type=jax.ShapeDtypeStruct((PACKED, 128), jnp.float32),
        mesh=mesh,
        scratch_types=[pltpu.VMEM((64, 128), jnp.float32)
                       for _ in range(2 * PNB)]
        + [pltpu.SemaphoreType.DMA for _ in range(2 * PNB)],
        compiler_params=params,
    )
    tableT = jnp.transpose(table)
    aux = lax.slice(tableT, (0, VOCAB - 128), (EMB, VOCAB))
    t128 = pack(tableT, aux)

    lookup = pl.kernel(
        _lookup_body,
        out_type=jax.ShapeDtypeStruct((L, EMB // 8, B // 128, 8, 128),
                                      jnp.float32),
        mesh=mesh,
        scratch_types=[
            pltpu.VMEM((L, 128), jnp.int32),
            pltpu.VMEM((L, 128), jnp.float32),
        ]
        + [pltpu.VMEM((128, EMB), jnp.float32) for _ in range(LNB)]
        + [pltpu.VMEM((8, 8, 128), jnp.float32) for _ in range(LNB)]
        + [pltpu.SemaphoreType.DMA for _ in range(2 * LNB)],
        compiler_params=pltpu.CompilerParams(use_tc_tiling_on_sc=False,
                                             needs_layout_passes=False),
    )
    t64 = jnp.reshape(t128, (VOCAB, EMB))
    out5 = lookup(inpT, t64, pe2)
    return jnp.transpose(out5, (2, 4, 0, 1, 3)).reshape(B, L, EMB)


def kernel(input, table):
    inpT = jnp.transpose(jnp.asarray(input, jnp.int32))
    pe2 = jnp.asarray(_PE2)
    return _sc_call(inpT, table, pe2)
